# Initial kernel scaffold; baseline (speedup 1.0000x reference)
import jax
import jax.numpy as jnp
from jax.experimental import pallas as pl


def kernel(x, ei0, ei1, ew0, ew1, ws00, wn00, b00, ws01, wn01, b01, ws10, wn10, b10, ws11, wn11, b11, wk0, bk0, wk1, bk1, wc1, bc1, wc2, bc2):
    raise NotImplementedError("write your pallas kernel here")


# speedup vs baseline: 4.3720x; 4.3720x over previous
import functools

import jax
import jax.numpy as jnp
from jax import lax
from jax.experimental import pallas as pl
from jax.experimental.pallas import tpu as pltpu
from jax.experimental.pallas import tpu_sc as plsc

N = 10000
D = 128
H = 128
C = 64
E = 160000

NC = 2
NS = 16
CH = 80
EPT = E // NS
NCHUNK = EPT // CH
GRP = 5
NSG = NCHUNK // GRP
R_BIG = 632
R_SMALL = N - (NS - 1) * R_BIG
R_EXTRA = R_BIG - R_SMALL
LANES = 16


def _build_segsum():
    scratch = [
        pltpu.VMEM((GRP, CH), jnp.int32),
        pltpu.VMEM((GRP, CH), jnp.int32),
        pltpu.VMEM((GRP, CH), jnp.float32),
        pltpu.VMEM((CH, H), jnp.float32),
        pltpu.VMEM((CH,), jnp.float32),
        pltpu.VMEM((CH + LANES,), jnp.float32),
        pltpu.VMEM_SHARED((N, H), jnp.float32),
        pltpu.VMEM_SHARED((N,), jnp.float32),
    ]

    mesh = plsc.VectorSubcoreMesh(core_axis_name="c", subcore_axis_name="s")

    @functools.partial(
        pl.kernel,
        out_type=jax.ShapeDtypeStruct((NC, N, H), jnp.float32),
        mesh=mesh,
        scratch_types=tuple(scratch),
    )
    def seg(x_hbm, src_hbm, dst_hbm, ew_hbm, out_hbm,
            src_v, dst_v, ew_v, rows_v, ones_v, db_v, acc, dacc):
        c = lax.axis_index("c")
        s = lax.axis_index("s")

        zero16 = jnp.zeros((LANES,), jnp.float32)
        one16 = jnp.ones((LANES,), jnp.float32)

        def init_row(i, carry):
            for j in range(H // LANES):
                rows_v[i, pl.ds(j * LANES, LANES)] = zero16
            return carry

        lax.fori_loop(0, CH, init_row, 0)
        for k in range(CH // LANES):
            ones_v[pl.ds(k * LANES, LANES)] = one16
        for k in range((CH + LANES) // LANES):
            db_v[pl.ds(k * LANES, LANES)] = zero16

        r0 = s * R_BIG

        def zero_span(base, total):
            nfull, rem = total // CH, total % CH
            for t in range(nfull):
                pltpu.sync_copy(rows_v, acc.at[pl.ds(base + t * CH, CH)])
                pltpu.sync_copy(db_v.at[pl.ds(0, CH)],
                                dacc.at[pl.ds(base + t * CH, CH)])
            if rem:
                pltpu.sync_copy(rows_v.at[pl.ds(0, rem)],
                                acc.at[pl.ds(base + nfull * CH, rem)])
                pltpu.sync_copy(db_v.at[pl.ds(0, rem)],
                                dacc.at[pl.ds(base + nfull * CH, rem)])

        zero_span(r0, R_SMALL)

        @pl.when(s < NS - 1)
        def _():
            zero_span(r0 + R_SMALL, R_EXTRA)

        plsc.subcore_barrier()

        def chunk(g, carry):
            pltpu.sync_copy(x_hbm.at[src_v.at[g]], rows_v)

            def scale_grp(eg, cc):
                w16 = ew_v[g, pl.ds(eg * LANES, LANES)]
                for l in range(LANES):
                    wb = jnp.full((LANES,), w16[l], dtype=jnp.float32)
                    i = eg * LANES + l
                    for j in range(H // LANES):
                        sl = rows_v[i, pl.ds(j * LANES, LANES)]
                        rows_v[i, pl.ds(j * LANES, LANES)] = sl * wb
                return cc

            lax.fori_loop(0, CH // LANES, scale_grp, 0)

            pltpu.sync_copy(rows_v, acc.at[dst_v.at[g]], add=True)
            pltpu.sync_copy(ones_v, dacc.at[dst_v.at[g]], add=True)
            return carry

        def supergrp(sg, carry):
            pltpu.sync_copy(src_hbm.at[c, s, sg], src_v)
            pltpu.sync_copy(dst_hbm.at[c, s, sg], dst_v)
            pltpu.sync_copy(ew_hbm.at[c, s, sg], ew_v)
            lax.fori_loop(0, GRP, chunk, 0)
            return carry

        lax.fori_loop(0, NSG, supergrp, 0)

        plsc.subcore_barrier()

        def norm_rows(nrows):
            def nr(i, carry):
                d16 = db_v[pl.ds(i, LANES)]
                dv = jnp.maximum(jnp.full((LANES,), d16[0], jnp.float32), 1.0)
                for j in range(H // LANES):
                    sl = rows_v[i, pl.ds(j * LANES, LANES)]
                    rows_v[i, pl.ds(j * LANES, LANES)] = sl / dv
                return carry
            lax.fori_loop(0, nrows, nr, 0)

        def norm_span(base, total):
            nfull, rem = total // CH, total % CH
            for t in range(nfull):
                b = base + t * CH
                pltpu.sync_copy(acc.at[pl.ds(b, CH)], rows_v)
                pltpu.sync_copy(dacc.at[pl.ds(b, CH)], db_v.at[pl.ds(0, CH)])
                norm_rows(CH)
                pltpu.sync_copy(rows_v, out_hbm.at[c, pl.ds(b, CH)])
            if rem:
                b = base + nfull * CH
                pltpu.sync_copy(acc.at[pl.ds(b, rem)], rows_v.at[pl.ds(0, rem)])
                pltpu.sync_copy(dacc.at[pl.ds(b, rem)], db_v.at[pl.ds(0, rem)])
                norm_rows(rem)
                pltpu.sync_copy(rows_v.at[pl.ds(0, rem)],
                                out_hbm.at[c, pl.ds(b, rem)])

        norm_span(r0, R_SMALL)

        @pl.when(s < NS - 1)
        def _():
            norm_span(r0 + R_SMALL, R_EXTRA)

    return seg


_segsum = _build_segsum()


BR = 1000


def _layer_body(h_ref, n0_ref, n1_ref, wk_ref, bk_ref,
                ws0_ref, ws1_ref, wn0_ref, wn1_ref, b0_ref, b1_ref, out_ref):
    h = h_ref[...]
    dot = functools.partial(jnp.dot, preferred_element_type=jnp.float32)
    z = (dot(h, ws0_ref[...] + ws1_ref[...]) + dot(n0_ref[...], wn0_ref[...])
         + dot(n1_ref[...], wn1_ref[...]) + (b0_ref[...] + b1_ref[...]))
    out_ref[...] = dot(h, wk_ref[...]) + bk_ref[...] + jnp.maximum(z, 0.0)


def _layer_head_body(h_ref, n0_ref, n1_ref, wk_ref, bk_ref,
                     ws0_ref, ws1_ref, wn0_ref, wn1_ref, b0_ref, b1_ref,
                     wc1_ref, bc1_ref, wc2_ref, bc2_ref, out_ref):
    h = h_ref[...]
    dot = functools.partial(jnp.dot, preferred_element_type=jnp.float32)
    z = (dot(h, ws0_ref[...] + ws1_ref[...]) + dot(n0_ref[...], wn0_ref[...])
         + dot(n1_ref[...], wn1_ref[...]) + (b0_ref[...] + b1_ref[...]))
    hn = dot(h, wk_ref[...]) + bk_ref[...] + jnp.maximum(z, 0.0)
    q = jnp.maximum(dot(hn, wc1_ref[...]) + bc1_ref[...], 0.0)
    out_ref[...] = dot(q, wc2_ref[...]) + bc2_ref[...]


def _row_spec(w):
    return pl.BlockSpec((BR, w), lambda i: (i, 0))


def _full_spec(shape):
    return pl.BlockSpec(shape, lambda i: tuple(0 for _ in shape))


def _layer_call(h, n0, n1, wk, bk, ws0, ws1, wn0, wn1, b0, b1):
    in_specs = [
        _row_spec(H), _row_spec(H), _row_spec(H),
        _full_spec((D, H)), _full_spec((1, H)),
        _full_spec((D, H)), _full_spec((D, H)),
        _full_spec((D, H)), _full_spec((D, H)),
        _full_spec((1, H)), _full_spec((1, H)),
    ]
    return pl.pallas_call(
        _layer_body,
        grid=(N // BR,),
        in_specs=in_specs,
        out_specs=_row_spec(H),
        out_shape=jax.ShapeDtypeStruct((N, H), jnp.float32),
    )(h, n0, n1, wk, bk.reshape(1, H), ws0, ws1, wn0, wn1,
      b0.reshape(1, H), b1.reshape(1, H))


def _layer_head_call(h, n0, n1, wk, bk, ws0, ws1, wn0, wn1, b0, b1,
                     wc1, bc1, wc2, bc2):
    in_specs = [
        _row_spec(H), _row_spec(H), _row_spec(H),
        _full_spec((H, H)), _full_spec((1, H)),
        _full_spec((H, H)), _full_spec((H, H)),
        _full_spec((H, H)), _full_spec((H, H)),
        _full_spec((1, H)), _full_spec((1, H)),
        _full_spec((H, H)), _full_spec((1, H)),
        _full_spec((H, C)), _full_spec((1, C)),
    ]
    return pl.pallas_call(
        _layer_head_body,
        grid=(N // BR,),
        in_specs=in_specs,
        out_specs=_row_spec(C),
        out_shape=jax.ShapeDtypeStruct((N, C), jnp.float32),
    )(h, n0, n1, wk, bk.reshape(1, H), ws0, ws1, wn0, wn1,
      b0.reshape(1, H), b1.reshape(1, H), wc1, bc1.reshape(1, H),
      wc2, bc2.reshape(1, C))


def kernel(x, ei0, ei1, ew0, ew1, ws00, wn00, b00, ws01, wn01, b01,
           ws10, wn10, b10, ws11, wn11, b11, wk0, bk0, wk1, bk1,
           wc1, bc1, wc2, bc2):
    srcs = jnp.stack([ei0[0], ei1[0]]).reshape(NC, NS, NSG, GRP, CH)
    dsts = jnp.stack([ei0[1], ei1[1]]).reshape(NC, NS, NSG, GRP, CH)
    ews = jnp.stack([ew0, ew1]).reshape(NC, NS, NSG, GRP, CH)

    n_l0 = _segsum(x, srcs, dsts, ews)
    h = _layer_call(x, n_l0[0], n_l0[1],
                    wk0, bk0, ws00, ws01, wn00, wn01, b00, b01)
    n_l1 = _segsum(h, srcs, dsts, ews)
    return _layer_head_call(h, n_l1[0], n_l1[1],
                            wk1, bk1, ws10, ws11, wn10, wn11, b10, b11,
                            wc1, bc1, wc2, bc2)


# speedup vs baseline: 6.8707x; 1.5715x over previous
import functools

import jax
import jax.numpy as jnp
from jax import lax
from jax.experimental import pallas as pl
from jax.experimental.pallas import tpu as pltpu
from jax.experimental.pallas import tpu_sc as plsc

N = 10000
D = 128
H = 128
C = 64
E = 160000

NC = 2
NS = 16
CH = 80
EPT = E // NS
NCHUNK = EPT // CH
GRP = 25
NSG = NCHUNK // GRP
PAIRS = (GRP - 1) // 2
R_BIG = 632
R_SMALL = N - (NS - 1) * R_BIG
R_EXTRA = R_BIG - R_SMALL
LANES = 16


def _build_segsum():
    scratch = [
        pltpu.VMEM((GRP, CH), jnp.int32),
        pltpu.VMEM((GRP, CH), jnp.int32),
        pltpu.VMEM((GRP, CH), jnp.float32),
        pltpu.VMEM((CH, H), jnp.float32),
        pltpu.VMEM((CH, H), jnp.float32),
        pltpu.VMEM((CH,), jnp.float32),
        pltpu.VMEM((CH + LANES,), jnp.float32),
        pltpu.VMEM_SHARED((N, H), jnp.float32),
        pltpu.VMEM_SHARED((N,), jnp.float32),
        pltpu.SemaphoreType.DMA,
        pltpu.SemaphoreType.DMA,
        pltpu.SemaphoreType.DMA,
        pltpu.SemaphoreType.DMA,
    ]

    mesh = plsc.VectorSubcoreMesh(core_axis_name="c", subcore_axis_name="s")

    @functools.partial(
        pl.kernel,
        out_type=jax.ShapeDtypeStruct((NC, N, H), jnp.float32),
        mesh=mesh,
        scratch_types=tuple(scratch),
    )
    def seg(x_hbm, src_hbm, dst_hbm, ew_hbm, out_hbm,
            src_v, dst_v, ew_v, rows_v, rows1_v, ones_v, db_v, acc, dacc,
            gsem0, gsem1, ssem0, ssem1):
        c = lax.axis_index("c")
        s = lax.axis_index("s")

        zero16 = jnp.zeros((LANES,), jnp.float32)
        one16 = jnp.ones((LANES,), jnp.float32)

        def init_row(i, carry):
            for j in range(H // LANES):
                rows_v[i, pl.ds(j * LANES, LANES)] = zero16
            return carry

        lax.fori_loop(0, CH, init_row, 0)
        for k in range(CH // LANES):
            ones_v[pl.ds(k * LANES, LANES)] = one16
        for k in range((CH + LANES) // LANES):
            db_v[pl.ds(k * LANES, LANES)] = zero16

        r0 = s * R_BIG

        def zero_span(base, total):
            nfull, rem = total // CH, total % CH
            for t in range(nfull):
                pltpu.sync_copy(rows_v, acc.at[pl.ds(base + t * CH, CH)])
                pltpu.sync_copy(db_v.at[pl.ds(0, CH)],
                                dacc.at[pl.ds(base + t * CH, CH)])
            if rem:
                pltpu.sync_copy(rows_v.at[pl.ds(0, rem)],
                                acc.at[pl.ds(base + nfull * CH, rem)])
                pltpu.sync_copy(db_v.at[pl.ds(0, rem)],
                                dacc.at[pl.ds(base + nfull * CH, rem)])

        zero_span(r0, R_SMALL)

        @pl.when(s < NS - 1)
        def _():
            zero_span(r0 + R_SMALL, R_EXTRA)

        plsc.subcore_barrier()

        def gather_start(cc, buf, sem):
            pltpu.async_copy(x_hbm.at[src_v.at[cc]], buf, sem)

        def gather_wait(buf, sem):
            pltpu.make_async_copy(x_hbm.at[src_v.at[0]], buf, sem).wait()

        def scatter_start(cc, buf, sem):
            pltpu.async_copy(buf, acc.at[dst_v.at[cc]], sem, add=True)
            pltpu.async_copy(ones_v, dacc.at[dst_v.at[cc]], sem, add=True)

        def scatter_wait(buf, sem):
            pltpu.make_async_copy(buf, acc.at[dst_v.at[0]], sem).wait()
            pltpu.make_async_copy(ones_v, dacc.at[dst_v.at[0]], sem).wait()

        def scale(buf, g):
            def scale_grp(eg, cc):
                w16 = ew_v[g, pl.ds(eg * LANES, LANES)]
                for l in range(LANES):
                    wb = jnp.full((LANES,), w16[l], dtype=jnp.float32)
                    i = eg * LANES + l
                    for j in range(H // LANES):
                        sl = buf[i, pl.ds(j * LANES, LANES)]
                        buf[i, pl.ds(j * LANES, LANES)] = sl * wb
                return cc

            lax.fori_loop(0, CH // LANES, scale_grp, 0)

        def pair(k, carry):
            @pl.when(k > 0)
            def _():
                scatter_wait(rows1_v, ssem1)
            gather_start(2 * k + 1, rows1_v, gsem1)
            gather_wait(rows_v, gsem0)
            scale(rows_v, 2 * k)
            scatter_start(2 * k, rows_v, ssem0)
            gather_wait(rows1_v, gsem1)
            scale(rows1_v, 2 * k + 1)
            scatter_start(2 * k + 1, rows1_v, ssem1)
            scatter_wait(rows_v, ssem0)
            gather_start(2 * k + 2, rows_v, gsem0)
            return carry

        def supergrp(sg, carry):
            pltpu.sync_copy(src_hbm.at[c, s, sg], src_v)
            pltpu.sync_copy(dst_hbm.at[c, s, sg], dst_v)
            pltpu.sync_copy(ew_hbm.at[c, s, sg], ew_v)

            gather_start(0, rows_v, gsem0)
            lax.fori_loop(0, PAIRS, pair, 0)
            scatter_wait(rows1_v, ssem1)
            gather_wait(rows_v, gsem0)
            scale(rows_v, GRP - 1)
            scatter_start(GRP - 1, rows_v, ssem0)
            scatter_wait(rows_v, ssem0)
            return carry

        lax.fori_loop(0, NSG, supergrp, 0)

        plsc.subcore_barrier()

        def norm_rows(nrows):
            def nr(i, carry):
                d16 = db_v[pl.ds(i, LANES)]
                dv = jnp.maximum(jnp.full((LANES,), d16[0], jnp.float32), 1.0)
                for j in range(H // LANES):
                    sl = rows_v[i, pl.ds(j * LANES, LANES)]
                    rows_v[i, pl.ds(j * LANES, LANES)] = sl / dv
                return carry
            lax.fori_loop(0, nrows, nr, 0)

        def norm_span(base, total):
            nfull, rem = total // CH, total % CH
            for t in range(nfull):
                b = base + t * CH
                pltpu.sync_copy(acc.at[pl.ds(b, CH)], rows_v)
                pltpu.sync_copy(dacc.at[pl.ds(b, CH)], db_v.at[pl.ds(0, CH)])
                norm_rows(CH)
                pltpu.sync_copy(rows_v, out_hbm.at[c, pl.ds(b, CH)])
            if rem:
                b = base + nfull * CH
                pltpu.sync_copy(acc.at[pl.ds(b, rem)], rows_v.at[pl.ds(0, rem)])
                pltpu.sync_copy(dacc.at[pl.ds(b, rem)], db_v.at[pl.ds(0, rem)])
                norm_rows(rem)
                pltpu.sync_copy(rows_v.at[pl.ds(0, rem)],
                                out_hbm.at[c, pl.ds(b, rem)])

        norm_span(r0, R_SMALL)

        @pl.when(s < NS - 1)
        def _():
            norm_span(r0 + R_SMALL, R_EXTRA)

    return seg


_segsum = _build_segsum()


BR = 1000


def _layer_body(h_ref, n0_ref, n1_ref, wk_ref, bk_ref,
                ws0_ref, ws1_ref, wn0_ref, wn1_ref, b0_ref, b1_ref, out_ref):
    h = h_ref[...]
    dot = functools.partial(jnp.dot, preferred_element_type=jnp.float32)
    z = (dot(h, ws0_ref[...] + ws1_ref[...]) + dot(n0_ref[...], wn0_ref[...])
         + dot(n1_ref[...], wn1_ref[...]) + (b0_ref[...] + b1_ref[...]))
    out_ref[...] = dot(h, wk_ref[...]) + bk_ref[...] + jnp.maximum(z, 0.0)


def _layer_head_body(h_ref, n0_ref, n1_ref, wk_ref, bk_ref,
                     ws0_ref, ws1_ref, wn0_ref, wn1_ref, b0_ref, b1_ref,
                     wc1_ref, bc1_ref, wc2_ref, bc2_ref, out_ref):
    h = h_ref[...]
    dot = functools.partial(jnp.dot, preferred_element_type=jnp.float32)
    z = (dot(h, ws0_ref[...] + ws1_ref[...]) + dot(n0_ref[...], wn0_ref[...])
         + dot(n1_ref[...], wn1_ref[...]) + (b0_ref[...] + b1_ref[...]))
    hn = dot(h, wk_ref[...]) + bk_ref[...] + jnp.maximum(z, 0.0)
    q = jnp.maximum(dot(hn, wc1_ref[...]) + bc1_ref[...], 0.0)
    out_ref[...] = dot(q, wc2_ref[...]) + bc2_ref[...]


def _row_spec(w):
    return pl.BlockSpec((BR, w), lambda i: (i, 0))


def _full_spec(shape):
    return pl.BlockSpec(shape, lambda i: tuple(0 for _ in shape))


def _layer_call(h, n0, n1, wk, bk, ws0, ws1, wn0, wn1, b0, b1):
    in_specs = [
        _row_spec(H), _row_spec(H), _row_spec(H),
        _full_spec((D, H)), _full_spec((1, H)),
        _full_spec((D, H)), _full_spec((D, H)),
        _full_spec((D, H)), _full_spec((D, H)),
        _full_spec((1, H)), _full_spec((1, H)),
    ]
    return pl.pallas_call(
        _layer_body,
        grid=(N // BR,),
        in_specs=in_specs,
        out_specs=_row_spec(H),
        out_shape=jax.ShapeDtypeStruct((N, H), jnp.float32),
    )(h, n0, n1, wk, bk.reshape(1, H), ws0, ws1, wn0, wn1,
      b0.reshape(1, H), b1.reshape(1, H))


def _layer_head_call(h, n0, n1, wk, bk, ws0, ws1, wn0, wn1, b0, b1,
                     wc1, bc1, wc2, bc2):
    in_specs = [
        _row_spec(H), _row_spec(H), _row_spec(H),
        _full_spec((H, H)), _full_spec((1, H)),
        _full_spec((H, H)), _full_spec((H, H)),
        _full_spec((H, H)), _full_spec((H, H)),
        _full_spec((1, H)), _full_spec((1, H)),
        _full_spec((H, H)), _full_spec((1, H)),
        _full_spec((H, C)), _full_spec((1, C)),
    ]
    return pl.pallas_call(
        _layer_head_body,
        grid=(N // BR,),
        in_specs=in_specs,
        out_specs=_row_spec(C),
        out_shape=jax.ShapeDtypeStruct((N, C), jnp.float32),
    )(h, n0, n1, wk, bk.reshape(1, H), ws0, ws1, wn0, wn1,
      b0.reshape(1, H), b1.reshape(1, H), wc1, bc1.reshape(1, H),
      wc2, bc2.reshape(1, C))


def kernel(x, ei0, ei1, ew0, ew1, ws00, wn00, b00, ws01, wn01, b01,
           ws10, wn10, b10, ws11, wn11, b11, wk0, bk0, wk1, bk1,
           wc1, bc1, wc2, bc2):
    srcs = jnp.stack([ei0[0], ei1[0]]).reshape(NC, NS, NSG, GRP, CH)
    dsts = jnp.stack([ei0[1], ei1[1]]).reshape(NC, NS, NSG, GRP, CH)
    ews = jnp.stack([ew0, ew1]).reshape(NC, NS, NSG, GRP, CH)

    n_l0 = _segsum(x, srcs, dsts, ews)
    h = _layer_call(x, n_l0[0], n_l0[1],
                    wk0, bk0, ws00, ws01, wn00, wn01, b00, b01)
    n_l1 = _segsum(h, srcs, dsts, ews)
    return _layer_head_call(h, n_l1[0], n_l1[1],
                            wk1, bk1, ws10, ws11, wn10, wn11, b10, b11,
                            wc1, bc1, wc2, bc2)


# speedup vs baseline: 8.2608x; 1.2023x over previous
import functools

import jax
import jax.numpy as jnp
from jax import lax
from jax.experimental import pallas as pl
from jax.experimental.pallas import tpu as pltpu
from jax.experimental.pallas import tpu_sc as plsc

N = 10000
D = 128
H = 128
C = 64
E = 160000

NC = 2
NS = 16
CH = 80
EPT = E // NS
NCHUNK = EPT // CH
GRP = 25
NSG = NCHUNK // GRP
TRIPS = (GRP - 1) // 3
R_BIG = 632
R_SMALL = N - (NS - 1) * R_BIG
R_EXTRA = R_BIG - R_SMALL
LANES = 16


def _build_segsum():
    scratch = [
        pltpu.VMEM((GRP, CH), jnp.int32),
        pltpu.VMEM((GRP, CH), jnp.int32),
        pltpu.VMEM((GRP, CH), jnp.float32),
        pltpu.VMEM((CH, H), jnp.float32),
        pltpu.VMEM((CH, H), jnp.float32),
        pltpu.VMEM((CH, H), jnp.float32),
        pltpu.VMEM((CH,), jnp.float32),
        pltpu.VMEM((CH + LANES,), jnp.float32),
        pltpu.VMEM_SHARED((N, H), jnp.float32),
        pltpu.VMEM_SHARED((N,), jnp.float32),
        pltpu.SemaphoreType.DMA,
        pltpu.SemaphoreType.DMA,
        pltpu.SemaphoreType.DMA,
        pltpu.SemaphoreType.DMA,
        pltpu.SemaphoreType.DMA,
        pltpu.SemaphoreType.DMA,
    ]

    mesh = plsc.VectorSubcoreMesh(core_axis_name="c", subcore_axis_name="s")

    @functools.partial(
        pl.kernel,
        out_type=jax.ShapeDtypeStruct((NC, N, H), jnp.float32),
        mesh=mesh,
        scratch_types=tuple(scratch),
    )
    def seg(x_hbm, src_hbm, dst_hbm, ew_hbm, out_hbm,
            src_v, dst_v, ew_v, rows_v, rows1_v, rows2_v, ones_v, db_v,
            acc, dacc, gsem0, gsem1, gsem2, ssem0, ssem1, ssem2):
        c = lax.axis_index("c")
        s = lax.axis_index("s")

        zero16 = jnp.zeros((LANES,), jnp.float32)
        one16 = jnp.ones((LANES,), jnp.float32)

        def init_row(i, carry):
            for j in range(H // LANES):
                rows_v[i, pl.ds(j * LANES, LANES)] = zero16
            return carry

        lax.fori_loop(0, CH, init_row, 0)
        for k in range(CH // LANES):
            ones_v[pl.ds(k * LANES, LANES)] = one16
        for k in range((CH + LANES) // LANES):
            db_v[pl.ds(k * LANES, LANES)] = zero16

        r0 = s * R_BIG

        def zero_span(base, total):
            nfull, rem = total // CH, total % CH
            for t in range(nfull):
                pltpu.sync_copy(rows_v, acc.at[pl.ds(base + t * CH, CH)])
                pltpu.sync_copy(db_v.at[pl.ds(0, CH)],
                                dacc.at[pl.ds(base + t * CH, CH)])
            if rem:
                pltpu.sync_copy(rows_v.at[pl.ds(0, rem)],
                                acc.at[pl.ds(base + nfull * CH, rem)])
                pltpu.sync_copy(db_v.at[pl.ds(0, rem)],
                                dacc.at[pl.ds(base + nfull * CH, rem)])

        zero_span(r0, R_SMALL)

        @pl.when(s < NS - 1)
        def _():
            zero_span(r0 + R_SMALL, R_EXTRA)

        plsc.subcore_barrier()

        def gather_start(cc, buf, sem):
            pltpu.async_copy(x_hbm.at[src_v.at[cc]], buf, sem)

        def gather_wait(buf, sem):
            pltpu.make_async_copy(x_hbm.at[src_v.at[0]], buf, sem).wait()

        def scatter_start(cc, buf, sem):
            pltpu.async_copy(buf, acc.at[dst_v.at[cc]], sem, add=True)
            pltpu.async_copy(ones_v, dacc.at[dst_v.at[cc]], sem, add=True)

        def scatter_wait(buf, sem):
            pltpu.make_async_copy(buf, acc.at[dst_v.at[0]], sem).wait()
            pltpu.make_async_copy(ones_v, dacc.at[dst_v.at[0]], sem).wait()

        def scale(buf, g):
            def scale_grp(eg, cc):
                w16 = ew_v[g, pl.ds(eg * LANES, LANES)]
                for l in range(LANES):
                    wb = jnp.full((LANES,), w16[l], dtype=jnp.float32)
                    i = eg * LANES + l
                    for j in range(H // LANES):
                        sl = buf[i, pl.ds(j * LANES, LANES)]
                        buf[i, pl.ds(j * LANES, LANES)] = sl * wb
                return cc

            lax.fori_loop(0, CH // LANES, scale_grp, 0)

        def triplet(k, carry):
            @pl.when(k > 0)
            def _():
                scatter_wait(rows1_v, ssem1)
            gather_start(3 * k + 1, rows1_v, gsem1)
            gather_wait(rows_v, gsem0)
            scale(rows_v, 3 * k)
            scatter_start(3 * k, rows_v, ssem0)
            @pl.when(k > 0)
            def _():
                scatter_wait(rows2_v, ssem2)
            gather_start(3 * k + 2, rows2_v, gsem2)
            gather_wait(rows1_v, gsem1)
            scale(rows1_v, 3 * k + 1)
            scatter_start(3 * k + 1, rows1_v, ssem1)
            scatter_wait(rows_v, ssem0)
            gather_start(3 * k + 3, rows_v, gsem0)
            gather_wait(rows2_v, gsem2)
            scale(rows2_v, 3 * k + 2)
            scatter_start(3 * k + 2, rows2_v, ssem2)
            return carry

        def supergrp(sg, carry):
            pltpu.sync_copy(src_hbm.at[c, s, sg], src_v)
            pltpu.sync_copy(dst_hbm.at[c, s, sg], dst_v)
            pltpu.sync_copy(ew_hbm.at[c, s, sg], ew_v)

            gather_start(0, rows_v, gsem0)
            lax.fori_loop(0, TRIPS, triplet, 0)
            scatter_wait(rows1_v, ssem1)
            gather_wait(rows_v, gsem0)
            scale(rows_v, GRP - 1)
            scatter_start(GRP - 1, rows_v, ssem0)
            scatter_wait(rows2_v, ssem2)
            scatter_wait(rows_v, ssem0)
            return carry

        lax.fori_loop(0, NSG, supergrp, 0)

        plsc.subcore_barrier()

        def norm_rows(nrows):
            def nr(i, carry):
                d16 = db_v[pl.ds(i, LANES)]
                dv = jnp.maximum(jnp.full((LANES,), d16[0], jnp.float32), 1.0)
                for j in range(H // LANES):
                    sl = rows_v[i, pl.ds(j * LANES, LANES)]
                    rows_v[i, pl.ds(j * LANES, LANES)] = sl / dv
                return carry
            lax.fori_loop(0, nrows, nr, 0)

        def norm_span(base, total):
            nfull, rem = total // CH, total % CH
            for t in range(nfull):
                b = base + t * CH
                pltpu.sync_copy(acc.at[pl.ds(b, CH)], rows_v)
                pltpu.sync_copy(dacc.at[pl.ds(b, CH)], db_v.at[pl.ds(0, CH)])
                norm_rows(CH)
                pltpu.sync_copy(rows_v, out_hbm.at[c, pl.ds(b, CH)])
            if rem:
                b = base + nfull * CH
                pltpu.sync_copy(acc.at[pl.ds(b, rem)], rows_v.at[pl.ds(0, rem)])
                pltpu.sync_copy(dacc.at[pl.ds(b, rem)], db_v.at[pl.ds(0, rem)])
                norm_rows(rem)
                pltpu.sync_copy(rows_v.at[pl.ds(0, rem)],
                                out_hbm.at[c, pl.ds(b, rem)])

        norm_span(r0, R_SMALL)

        @pl.when(s < NS - 1)
        def _():
            norm_span(r0 + R_SMALL, R_EXTRA)

    return seg


_segsum = _build_segsum()


BR = 1000


def _layer_body(h_ref, n0_ref, n1_ref, wk_ref, bk_ref,
                ws0_ref, ws1_ref, wn0_ref, wn1_ref, b0_ref, b1_ref, out_ref):
    h = h_ref[...]
    dot = functools.partial(jnp.dot, preferred_element_type=jnp.float32)
    z = (dot(h, ws0_ref[...] + ws1_ref[...]) + dot(n0_ref[...], wn0_ref[...])
         + dot(n1_ref[...], wn1_ref[...]) + (b0_ref[...] + b1_ref[...]))
    out_ref[...] = dot(h, wk_ref[...]) + bk_ref[...] + jnp.maximum(z, 0.0)


def _layer_head_body(h_ref, n0_ref, n1_ref, wk_ref, bk_ref,
                     ws0_ref, ws1_ref, wn0_ref, wn1_ref, b0_ref, b1_ref,
                     wc1_ref, bc1_ref, wc2_ref, bc2_ref, out_ref):
    h = h_ref[...]
    dot = functools.partial(jnp.dot, preferred_element_type=jnp.float32)
    z = (dot(h, ws0_ref[...] + ws1_ref[...]) + dot(n0_ref[...], wn0_ref[...])
         + dot(n1_ref[...], wn1_ref[...]) + (b0_ref[...] + b1_ref[...]))
    hn = dot(h, wk_ref[...]) + bk_ref[...] + jnp.maximum(z, 0.0)
    q = jnp.maximum(dot(hn, wc1_ref[...]) + bc1_ref[...], 0.0)
    out_ref[...] = dot(q, wc2_ref[...]) + bc2_ref[...]


def _row_spec(w):
    return pl.BlockSpec((BR, w), lambda i: (i, 0))


def _full_spec(shape):
    return pl.BlockSpec(shape, lambda i: tuple(0 for _ in shape))


def _layer_call(h, n0, n1, wk, bk, ws0, ws1, wn0, wn1, b0, b1):
    in_specs = [
        _row_spec(H), _row_spec(H), _row_spec(H),
        _full_spec((D, H)), _full_spec((1, H)),
        _full_spec((D, H)), _full_spec((D, H)),
        _full_spec((D, H)), _full_spec((D, H)),
        _full_spec((1, H)), _full_spec((1, H)),
    ]
    return pl.pallas_call(
        _layer_body,
        grid=(N // BR,),
        in_specs=in_specs,
        out_specs=_row_spec(H),
        out_shape=jax.ShapeDtypeStruct((N, H), jnp.float32),
    )(h, n0, n1, wk, bk.reshape(1, H), ws0, ws1, wn0, wn1,
      b0.reshape(1, H), b1.reshape(1, H))


def _layer_head_call(h, n0, n1, wk, bk, ws0, ws1, wn0, wn1, b0, b1,
                     wc1, bc1, wc2, bc2):
    in_specs = [
        _row_spec(H), _row_spec(H), _row_spec(H),
        _full_spec((H, H)), _full_spec((1, H)),
        _full_spec((H, H)), _full_spec((H, H)),
        _full_spec((H, H)), _full_spec((H, H)),
        _full_spec((1, H)), _full_spec((1, H)),
        _full_spec((H, H)), _full_spec((1, H)),
        _full_spec((H, C)), _full_spec((1, C)),
    ]
    return pl.pallas_call(
        _layer_head_body,
        grid=(N // BR,),
        in_specs=in_specs,
        out_specs=_row_spec(C),
        out_shape=jax.ShapeDtypeStruct((N, C), jnp.float32),
    )(h, n0, n1, wk, bk.reshape(1, H), ws0, ws1, wn0, wn1,
      b0.reshape(1, H), b1.reshape(1, H), wc1, bc1.reshape(1, H),
      wc2, bc2.reshape(1, C))


def kernel(x, ei0, ei1, ew0, ew1, ws00, wn00, b00, ws01, wn01, b01,
           ws10, wn10, b10, ws11, wn11, b11, wk0, bk0, wk1, bk1,
           wc1, bc1, wc2, bc2):
    srcs = jnp.stack([ei0[0], ei1[0]]).reshape(NC, NS, NSG, GRP, CH)
    dsts = jnp.stack([ei0[1], ei1[1]]).reshape(NC, NS, NSG, GRP, CH)
    ews = jnp.stack([ew0, ew1]).reshape(NC, NS, NSG, GRP, CH)

    n_l0 = _segsum(x, srcs, dsts, ews)
    h = _layer_call(x, n_l0[0], n_l0[1],
                    wk0, bk0, ws00, ws01, wn00, wn01, b00, b01)
    n_l1 = _segsum(h, srcs, dsts, ews)
    return _layer_head_call(h, n_l1[0], n_l1[1],
                            wk1, bk1, ws10, ws11, wn10, wn11, b10, b11,
                            wc1, bc1, wc2, bc2)


# speedup vs baseline: 8.4983x; 1.0288x over previous
import functools

import jax
import jax.numpy as jnp
from jax import lax
from jax.experimental import pallas as pl
from jax.experimental.pallas import tpu as pltpu
from jax.experimental.pallas import tpu_sc as plsc

N = 10000
D = 128
H = 128
C = 64
E = 160000

NC = 2
NS = 16
CH = 80
EPT = E // NS
NCHUNK = EPT // CH
GRP = 25
NSG = NCHUNK // GRP
TRIPS = (GRP - 1) // 3
R_BIG = 632
R_SMALL = N - (NS - 1) * R_BIG
R_EXTRA = R_BIG - R_SMALL
LANES = 16


def _build_segsum(with_deg):
    out_type = [jax.ShapeDtypeStruct((NC, N, H), jnp.float32)]
    if with_deg:
        out_type.append(jax.ShapeDtypeStruct((NC * N,), jnp.float32))
    scratch = [
        pltpu.VMEM((GRP, CH), jnp.int32),
        pltpu.VMEM((GRP, CH), jnp.int32),
        pltpu.VMEM((GRP, CH), jnp.float32),
        pltpu.VMEM((CH, H), jnp.float32),
        pltpu.VMEM((CH, H), jnp.float32),
        pltpu.VMEM((CH, H), jnp.float32),
        pltpu.VMEM_SHARED((N, H), jnp.float32),
        pltpu.SemaphoreType.DMA,
        pltpu.SemaphoreType.DMA,
        pltpu.SemaphoreType.DMA,
        pltpu.SemaphoreType.DMA,
        pltpu.SemaphoreType.DMA,
        pltpu.SemaphoreType.DMA,
    ]
    if with_deg:
        scratch.append(pltpu.VMEM((CH,), jnp.float32))
        scratch.append(pltpu.VMEM_SHARED((N,), jnp.float32))

    mesh = plsc.VectorSubcoreMesh(core_axis_name="c", subcore_axis_name="s")

    @functools.partial(
        pl.kernel,
        out_type=tuple(out_type) if with_deg else out_type[0],
        mesh=mesh,
        scratch_types=tuple(scratch),
    )
    def seg(x_hbm, src_hbm, dst_hbm, ew_hbm, *refs):
        if with_deg:
            (s_hbm, deg_hbm, src_v, dst_v, ew_v, rows_v, rows1_v, rows2_v,
             acc, gsem0, gsem1, gsem2, ssem0, ssem1, ssem2,
             ones_v, dacc) = refs
        else:
            (s_hbm, src_v, dst_v, ew_v, rows_v, rows1_v, rows2_v,
             acc, gsem0, gsem1, gsem2, ssem0, ssem1, ssem2) = refs
        c = lax.axis_index("c")
        s = lax.axis_index("s")

        zero16 = jnp.zeros((LANES,), jnp.float32)
        one16 = jnp.ones((LANES,), jnp.float32)

        def init_row(i, carry):
            for j in range(H // LANES):
                rows_v[i, pl.ds(j * LANES, LANES)] = zero16
            return carry

        lax.fori_loop(0, CH, init_row, 0)
        if with_deg:
            for k in range(CH // LANES):
                ones_v[pl.ds(k * LANES, LANES)] = zero16

        r0 = s * R_BIG

        def zero_span(base, total):
            nfull, rem = total // CH, total % CH
            for t in range(nfull):
                pltpu.sync_copy(rows_v, acc.at[pl.ds(base + t * CH, CH)])
                if with_deg:
                    pltpu.sync_copy(ones_v, dacc.at[pl.ds(base + t * CH, CH)])
            if rem:
                pltpu.sync_copy(rows_v.at[pl.ds(0, rem)],
                                acc.at[pl.ds(base + nfull * CH, rem)])
                if with_deg:
                    pltpu.sync_copy(ones_v.at[pl.ds(0, rem)],
                                    dacc.at[pl.ds(base + nfull * CH, rem)])

        zero_span(r0, R_SMALL)

        @pl.when(s < NS - 1)
        def _():
            zero_span(r0 + R_SMALL, R_EXTRA)

        if with_deg:
            for k in range(CH // LANES):
                ones_v[pl.ds(k * LANES, LANES)] = one16

        plsc.subcore_barrier()

        def gather_start(cc, buf, sem):
            pltpu.async_copy(x_hbm.at[src_v.at[cc]], buf, sem)

        def gather_wait(buf, sem):
            pltpu.make_async_copy(x_hbm.at[src_v.at[0]], buf, sem).wait()

        def scatter_start(cc, buf, sem):
            pltpu.async_copy(buf, acc.at[dst_v.at[cc]], sem, add=True)
            if with_deg:
                pltpu.async_copy(ones_v, dacc.at[dst_v.at[cc]], sem, add=True)

        def scatter_wait(buf, sem):
            pltpu.make_async_copy(buf, acc.at[dst_v.at[0]], sem).wait()
            if with_deg:
                pltpu.make_async_copy(ones_v, dacc.at[dst_v.at[0]], sem).wait()

        def scale(buf, g):
            def scale_grp(eg, cc):
                w16 = ew_v[g, pl.ds(eg * LANES, LANES)]
                for l in range(LANES):
                    wb = jnp.full((LANES,), w16[l], dtype=jnp.float32)
                    i = eg * LANES + l
                    for j in range(H // LANES):
                        sl = buf[i, pl.ds(j * LANES, LANES)]
                        buf[i, pl.ds(j * LANES, LANES)] = sl * wb
                return cc

            lax.fori_loop(0, CH // LANES, scale_grp, 0)

        def triplet(k, carry):
            @pl.when(k > 0)
            def _():
                scatter_wait(rows1_v, ssem1)
            gather_start(3 * k + 1, rows1_v, gsem1)
            gather_wait(rows_v, gsem0)
            scale(rows_v, 3 * k)
            scatter_start(3 * k, rows_v, ssem0)
            @pl.when(k > 0)
            def _():
                scatter_wait(rows2_v, ssem2)
            gather_start(3 * k + 2, rows2_v, gsem2)
            gather_wait(rows1_v, gsem1)
            scale(rows1_v, 3 * k + 1)
            scatter_start(3 * k + 1, rows1_v, ssem1)
            scatter_wait(rows_v, ssem0)
            gather_start(3 * k + 3, rows_v, gsem0)
            gather_wait(rows2_v, gsem2)
            scale(rows2_v, 3 * k + 2)
            scatter_start(3 * k + 2, rows2_v, ssem2)
            return carry

        def supergrp(sg, carry):
            pltpu.sync_copy(src_hbm.at[c, s, sg], src_v)
            pltpu.sync_copy(dst_hbm.at[c, s, sg], dst_v)
            pltpu.sync_copy(ew_hbm.at[c, s, sg], ew_v)

            gather_start(0, rows_v, gsem0)
            lax.fori_loop(0, TRIPS, triplet, 0)
            scatter_wait(rows1_v, ssem1)
            gather_wait(rows_v, gsem0)
            scale(rows_v, GRP - 1)
            scatter_start(GRP - 1, rows_v, ssem0)
            scatter_wait(rows2_v, ssem2)
            scatter_wait(rows_v, ssem0)
            return carry

        lax.fori_loop(0, NSG, supergrp, 0)

        plsc.subcore_barrier()

        def write_span(base, total):
            pltpu.sync_copy(acc.at[pl.ds(base, total)],
                            s_hbm.at[c, pl.ds(base, total)])
            if with_deg:
                nfull, rem = total // CH, total % CH
                for t in range(nfull):
                    b = base + t * CH
                    pltpu.sync_copy(dacc.at[pl.ds(b, CH)], ones_v)
                    pltpu.sync_copy(ones_v, deg_hbm.at[pl.ds(c * N + b, CH)])
                if rem:
                    b = base + nfull * CH
                    pltpu.sync_copy(dacc.at[pl.ds(b, rem)],
                                    ones_v.at[pl.ds(0, rem)])
                    pltpu.sync_copy(ones_v.at[pl.ds(0, rem)],
                                    deg_hbm.at[pl.ds(c * N + b, rem)])

        write_span(r0, R_SMALL)

        @pl.when(s < NS - 1)
        def _():
            write_span(r0 + R_SMALL, R_EXTRA)

    return seg


_segsum_deg = _build_segsum(True)
_segsum_nodeg = _build_segsum(False)


BR = 1000


def _layer_body(h_ref, s0_ref, s1_ref, d0_ref, d1_ref, wk_ref, bk_ref,
                ws0_ref, ws1_ref, wn0_ref, wn1_ref, b0_ref, b1_ref, out_ref):
    h = h_ref[...]
    n0 = s0_ref[...] / jnp.maximum(d0_ref[...], 1.0)
    n1 = s1_ref[...] / jnp.maximum(d1_ref[...], 1.0)
    dot = functools.partial(jnp.dot, preferred_element_type=jnp.float32)
    z = (dot(h, ws0_ref[...] + ws1_ref[...]) + dot(n0, wn0_ref[...])
         + dot(n1, wn1_ref[...]) + (b0_ref[...] + b1_ref[...]))
    out_ref[...] = dot(h, wk_ref[...]) + bk_ref[...] + jnp.maximum(z, 0.0)


def _layer_head_body(h_ref, s0_ref, s1_ref, d0_ref, d1_ref, wk_ref, bk_ref,
                     ws0_ref, ws1_ref, wn0_ref, wn1_ref, b0_ref, b1_ref,
                     wc1_ref, bc1_ref, wc2_ref, bc2_ref, out_ref):
    h = h_ref[...]
    n0 = s0_ref[...] / jnp.maximum(d0_ref[...], 1.0)
    n1 = s1_ref[...] / jnp.maximum(d1_ref[...], 1.0)
    dot = functools.partial(jnp.dot, preferred_element_type=jnp.float32)
    z = (dot(h, ws0_ref[...] + ws1_ref[...]) + dot(n0, wn0_ref[...])
         + dot(n1, wn1_ref[...]) + (b0_ref[...] + b1_ref[...]))
    hn = dot(h, wk_ref[...]) + bk_ref[...] + jnp.maximum(z, 0.0)
    q = jnp.maximum(dot(hn, wc1_ref[...]) + bc1_ref[...], 0.0)
    out_ref[...] = dot(q, wc2_ref[...]) + bc2_ref[...]


def _row_spec(w):
    return pl.BlockSpec((BR, w), lambda i: (i, 0))


def _full_spec(shape):
    return pl.BlockSpec(shape, lambda i: tuple(0 for _ in shape))


def _layer_call(h, s0, s1, d0, d1, wk, bk, ws0, ws1, wn0, wn1, b0, b1):
    in_specs = [
        _row_spec(H), _row_spec(H), _row_spec(H),
        _row_spec(1), _row_spec(1),
        _full_spec((D, H)), _full_spec((1, H)),
        _full_spec((D, H)), _full_spec((D, H)),
        _full_spec((D, H)), _full_spec((D, H)),
        _full_spec((1, H)), _full_spec((1, H)),
    ]
    return pl.pallas_call(
        _layer_body,
        grid=(N // BR,),
        in_specs=in_specs,
        out_specs=_row_spec(H),
        out_shape=jax.ShapeDtypeStruct((N, H), jnp.float32),
    )(h, s0, s1, d0, d1, wk, bk.reshape(1, H), ws0, ws1, wn0, wn1,
      b0.reshape(1, H), b1.reshape(1, H))


def _layer_head_call(h, s0, s1, d0, d1, wk, bk, ws0, ws1, wn0, wn1, b0, b1,
                     wc1, bc1, wc2, bc2):
    in_specs = [
        _row_spec(H), _row_spec(H), _row_spec(H),
        _row_spec(1), _row_spec(1),
        _full_spec((H, H)), _full_spec((1, H)),
        _full_spec((H, H)), _full_spec((H, H)),
        _full_spec((H, H)), _full_spec((H, H)),
        _full_spec((1, H)), _full_spec((1, H)),
        _full_spec((H, H)), _full_spec((1, H)),
        _full_spec((H, C)), _full_spec((1, C)),
    ]
    return pl.pallas_call(
        _layer_head_body,
        grid=(N // BR,),
        in_specs=in_specs,
        out_specs=_row_spec(C),
        out_shape=jax.ShapeDtypeStruct((N, C), jnp.float32),
    )(h, s0, s1, d0, d1, wk, bk.reshape(1, H), ws0, ws1, wn0, wn1,
      b0.reshape(1, H), b1.reshape(1, H), wc1, bc1.reshape(1, H),
      wc2, bc2.reshape(1, C))


def kernel(x, ei0, ei1, ew0, ew1, ws00, wn00, b00, ws01, wn01, b01,
           ws10, wn10, b10, ws11, wn11, b11, wk0, bk0, wk1, bk1,
           wc1, bc1, wc2, bc2):
    srcs = jnp.stack([ei0[0], ei1[0]]).reshape(NC, NS, NSG, GRP, CH)
    dsts = jnp.stack([ei0[1], ei1[1]]).reshape(NC, NS, NSG, GRP, CH)
    ews = jnp.stack([ew0, ew1]).reshape(NC, NS, NSG, GRP, CH)

    s_l0, degs = _segsum_deg(x, srcs, dsts, ews)
    d0 = degs[:N].reshape(N, 1)
    d1 = degs[N:].reshape(N, 1)
    h = _layer_call(x, s_l0[0], s_l0[1], d0, d1,
                    wk0, bk0, ws00, ws01, wn00, wn01, b00, b01)
    s_l1 = _segsum_nodeg(h, srcs, dsts, ews)
    return _layer_head_call(h, s_l1[0], s_l1[1], d0, d1,
                            wk1, bk1, ws10, ws11, wn10, wn11, b10, b11,
                            wc1, bc1, wc2, bc2)


# speedup vs baseline: 8.6149x; 1.0137x over previous
import functools

import jax
import jax.numpy as jnp
from jax import lax
from jax.experimental import pallas as pl
from jax.experimental.pallas import tpu as pltpu
from jax.experimental.pallas import tpu_sc as plsc

N = 10000
D = 128
H = 128
C = 64
E = 160000

NC = 2
NS = 16
CH = 80
EPT = E // NS
NCHUNK = EPT // CH
GRP = 25
NSG = NCHUNK // GRP
TRIPS = (GRP - 1) // 3
R_BIG = 632
R_SMALL = N - (NS - 1) * R_BIG
R_EXTRA = R_BIG - R_SMALL
LANES = 16


def _build_segsum(with_deg):
    out_type = [jax.ShapeDtypeStruct((NC, N, H), jnp.float32)]
    if with_deg:
        out_type.append(jax.ShapeDtypeStruct((NC * N,), jnp.float32))
    scratch = [
        pltpu.VMEM((GRP, CH), jnp.int32),
        pltpu.VMEM((GRP, CH), jnp.int32),
        pltpu.VMEM((GRP, CH), jnp.float32),
        pltpu.VMEM((CH, H), jnp.float32),
        pltpu.VMEM((CH, H), jnp.float32),
        pltpu.VMEM((CH, H), jnp.float32),
        pltpu.VMEM_SHARED((N, H), jnp.float32),
        pltpu.SemaphoreType.DMA,
        pltpu.SemaphoreType.DMA,
        pltpu.SemaphoreType.DMA,
        pltpu.SemaphoreType.DMA,
        pltpu.SemaphoreType.DMA,
        pltpu.SemaphoreType.DMA,
    ]
    if with_deg:
        scratch.append(pltpu.VMEM((CH,), jnp.float32))
        scratch.append(pltpu.VMEM_SHARED((N,), jnp.float32))

    mesh = plsc.VectorSubcoreMesh(core_axis_name="c", subcore_axis_name="s")

    @functools.partial(
        pl.kernel,
        out_type=tuple(out_type) if with_deg else out_type[0],
        mesh=mesh,
        scratch_types=tuple(scratch),
    )
    def seg(x_hbm, src_hbm, dst_hbm, ew_hbm, *refs):
        if with_deg:
            (s_hbm, deg_hbm, src_v, dst_v, ew_v, rows_v, rows1_v, rows2_v,
             acc, gsem0, gsem1, gsem2, ssem0, ssem1, ssem2,
             ones_v, dacc) = refs
        else:
            (s_hbm, src_v, dst_v, ew_v, rows_v, rows1_v, rows2_v,
             acc, gsem0, gsem1, gsem2, ssem0, ssem1, ssem2) = refs
        c = lax.axis_index("c")
        s = lax.axis_index("s")

        zero16 = jnp.zeros((LANES,), jnp.float32)
        one16 = jnp.ones((LANES,), jnp.float32)

        def init_row(i, carry):
            for j in range(H // LANES):
                rows_v[i, pl.ds(j * LANES, LANES)] = zero16
            return carry

        lax.fori_loop(0, CH, init_row, 0)
        if with_deg:
            for k in range(CH // LANES):
                ones_v[pl.ds(k * LANES, LANES)] = zero16

        r0 = s * R_BIG

        def zero_span(base, total):
            nfull, rem = total // CH, total % CH
            for t in range(nfull):
                pltpu.sync_copy(rows_v, acc.at[pl.ds(base + t * CH, CH)])
                if with_deg:
                    pltpu.sync_copy(ones_v, dacc.at[pl.ds(base + t * CH, CH)])
            if rem:
                pltpu.sync_copy(rows_v.at[pl.ds(0, rem)],
                                acc.at[pl.ds(base + nfull * CH, rem)])
                if with_deg:
                    pltpu.sync_copy(ones_v.at[pl.ds(0, rem)],
                                    dacc.at[pl.ds(base + nfull * CH, rem)])

        zero_span(r0, R_SMALL)

        @pl.when(s < NS - 1)
        def _():
            zero_span(r0 + R_SMALL, R_EXTRA)

        if with_deg:
            for k in range(CH // LANES):
                ones_v[pl.ds(k * LANES, LANES)] = one16

        plsc.subcore_barrier()

        def gather_start(cc, buf, sem):
            pltpu.async_copy(x_hbm.at[src_v.at[cc]], buf, sem)

        def gather_wait(buf, sem):
            pltpu.make_async_copy(x_hbm.at[src_v.at[0]], buf, sem).wait()

        def scatter_start(cc, buf, sem):
            pltpu.async_copy(buf, acc.at[dst_v.at[cc]], sem, add=True)
            if with_deg:
                pltpu.async_copy(ones_v, dacc.at[dst_v.at[cc]], sem, add=True)

        def scatter_wait(buf, sem):
            pltpu.make_async_copy(buf, acc.at[dst_v.at[0]], sem).wait()
            if with_deg:
                pltpu.make_async_copy(ones_v, dacc.at[dst_v.at[0]], sem).wait()

        def scale(buf, g):
            def scale_grp(eg, cc):
                w16 = ew_v[g, pl.ds(eg * LANES, LANES)]
                for l in range(LANES):
                    wb = jnp.full((LANES,), w16[l], dtype=jnp.float32)
                    i = eg * LANES + l
                    for j in range(H // LANES):
                        sl = buf[i, pl.ds(j * LANES, LANES)]
                        buf[i, pl.ds(j * LANES, LANES)] = sl * wb
                return cc

            lax.fori_loop(0, CH // LANES, scale_grp, 0)

        def triplet(k, carry):
            @pl.when(k > 0)
            def _():
                scatter_wait(rows1_v, ssem1)
            gather_start(3 * k + 1, rows1_v, gsem1)
            gather_wait(rows_v, gsem0)
            scale(rows_v, 3 * k)
            scatter_start(3 * k, rows_v, ssem0)
            @pl.when(k > 0)
            def _():
                scatter_wait(rows2_v, ssem2)
            gather_start(3 * k + 2, rows2_v, gsem2)
            gather_wait(rows1_v, gsem1)
            scale(rows1_v, 3 * k + 1)
            scatter_start(3 * k + 1, rows1_v, ssem1)
            scatter_wait(rows_v, ssem0)
            gather_start(3 * k + 3, rows_v, gsem0)
            gather_wait(rows2_v, gsem2)
            scale(rows2_v, 3 * k + 2)
            scatter_start(3 * k + 2, rows2_v, ssem2)
            return carry

        def supergrp(sg, carry):
            pltpu.sync_copy(src_hbm.at[c, s, sg], src_v)
            pltpu.sync_copy(dst_hbm.at[c, s, sg], dst_v)
            pltpu.sync_copy(ew_hbm.at[c, s, sg], ew_v)

            gather_start(0, rows_v, gsem0)
            lax.fori_loop(0, TRIPS, triplet, 0)
            scatter_wait(rows1_v, ssem1)
            gather_wait(rows_v, gsem0)
            scale(rows_v, GRP - 1)
            scatter_start(GRP - 1, rows_v, ssem0)
            scatter_wait(rows2_v, ssem2)
            scatter_wait(rows_v, ssem0)
            return carry

        lax.fori_loop(0, NSG, supergrp, 0)

        plsc.subcore_barrier()

        def write_span(base, total):
            pltpu.sync_copy(acc.at[pl.ds(base, total)],
                            s_hbm.at[c, pl.ds(base, total)])
            if with_deg:
                nfull, rem = total // CH, total % CH
                for t in range(nfull):
                    b = base + t * CH
                    pltpu.sync_copy(dacc.at[pl.ds(b, CH)], ones_v)
                    pltpu.sync_copy(ones_v, deg_hbm.at[pl.ds(c * N + b, CH)])
                if rem:
                    b = base + nfull * CH
                    pltpu.sync_copy(dacc.at[pl.ds(b, rem)],
                                    ones_v.at[pl.ds(0, rem)])
                    pltpu.sync_copy(ones_v.at[pl.ds(0, rem)],
                                    deg_hbm.at[pl.ds(c * N + b, rem)])

        write_span(r0, R_SMALL)

        @pl.when(s < NS - 1)
        def _():
            write_span(r0 + R_SMALL, R_EXTRA)

    return seg


_segsum_deg = _build_segsum(True)
_segsum_nodeg = _build_segsum(False)


BR = 2000


def _layer_body(h_ref, s0_ref, s1_ref, d0_ref, d1_ref, wk_ref, bk_ref,
                ws0_ref, ws1_ref, wn0_ref, wn1_ref, b0_ref, b1_ref, out_ref):
    h = h_ref[...]
    n0 = s0_ref[...] / jnp.maximum(d0_ref[...], 1.0)
    n1 = s1_ref[...] / jnp.maximum(d1_ref[...], 1.0)
    dot = functools.partial(jnp.dot, preferred_element_type=jnp.float32)
    z = (dot(h, ws0_ref[...] + ws1_ref[...]) + dot(n0, wn0_ref[...])
         + dot(n1, wn1_ref[...]) + (b0_ref[...] + b1_ref[...]))
    out_ref[...] = dot(h, wk_ref[...]) + bk_ref[...] + jnp.maximum(z, 0.0)


def _layer_head_body(h_ref, s0_ref, s1_ref, d0_ref, d1_ref, wk_ref, bk_ref,
                     ws0_ref, ws1_ref, wn0_ref, wn1_ref, b0_ref, b1_ref,
                     wc1_ref, bc1_ref, wc2_ref, bc2_ref, out_ref):
    h = h_ref[...]
    n0 = s0_ref[...] / jnp.maximum(d0_ref[...], 1.0)
    n1 = s1_ref[...] / jnp.maximum(d1_ref[...], 1.0)
    dot = functools.partial(jnp.dot, preferred_element_type=jnp.float32)
    z = (dot(h, ws0_ref[...] + ws1_ref[...]) + dot(n0, wn0_ref[...])
         + dot(n1, wn1_ref[...]) + (b0_ref[...] + b1_ref[...]))
    hn = dot(h, wk_ref[...]) + bk_ref[...] + jnp.maximum(z, 0.0)
    q = jnp.maximum(dot(hn, wc1_ref[...]) + bc1_ref[...], 0.0)
    out_ref[...] = dot(q, wc2_ref[...]) + bc2_ref[...]


def _row_spec(w):
    return pl.BlockSpec((BR, w), lambda i: (i, 0))


def _full_spec(shape):
    return pl.BlockSpec(shape, lambda i: tuple(0 for _ in shape))


def _layer_call(h, s0, s1, d0, d1, wk, bk, ws0, ws1, wn0, wn1, b0, b1):
    in_specs = [
        _row_spec(H), _row_spec(H), _row_spec(H),
        _row_spec(1), _row_spec(1),
        _full_spec((D, H)), _full_spec((1, H)),
        _full_spec((D, H)), _full_spec((D, H)),
        _full_spec((D, H)), _full_spec((D, H)),
        _full_spec((1, H)), _full_spec((1, H)),
    ]
    return pl.pallas_call(
        _layer_body,
        grid=(N // BR,),
        in_specs=in_specs,
        out_specs=_row_spec(H),
        out_shape=jax.ShapeDtypeStruct((N, H), jnp.float32),
    )(h, s0, s1, d0, d1, wk, bk.reshape(1, H), ws0, ws1, wn0, wn1,
      b0.reshape(1, H), b1.reshape(1, H))


def _layer_head_call(h, s0, s1, d0, d1, wk, bk, ws0, ws1, wn0, wn1, b0, b1,
                     wc1, bc1, wc2, bc2):
    in_specs = [
        _row_spec(H), _row_spec(H), _row_spec(H),
        _row_spec(1), _row_spec(1),
        _full_spec((H, H)), _full_spec((1, H)),
        _full_spec((H, H)), _full_spec((H, H)),
        _full_spec((H, H)), _full_spec((H, H)),
        _full_spec((1, H)), _full_spec((1, H)),
        _full_spec((H, H)), _full_spec((1, H)),
        _full_spec((H, C)), _full_spec((1, C)),
    ]
    return pl.pallas_call(
        _layer_head_body,
        grid=(N // BR,),
        in_specs=in_specs,
        out_specs=_row_spec(C),
        out_shape=jax.ShapeDtypeStruct((N, C), jnp.float32),
    )(h, s0, s1, d0, d1, wk, bk.reshape(1, H), ws0, ws1, wn0, wn1,
      b0.reshape(1, H), b1.reshape(1, H), wc1, bc1.reshape(1, H),
      wc2, bc2.reshape(1, C))


def kernel(x, ei0, ei1, ew0, ew1, ws00, wn00, b00, ws01, wn01, b01,
           ws10, wn10, b10, ws11, wn11, b11, wk0, bk0, wk1, bk1,
           wc1, bc1, wc2, bc2):
    srcs = jnp.stack([ei0[0], ei1[0]]).reshape(NC, NS, NSG, GRP, CH)
    dsts = jnp.stack([ei0[1], ei1[1]]).reshape(NC, NS, NSG, GRP, CH)
    ews = jnp.stack([ew0, ew1]).reshape(NC, NS, NSG, GRP, CH)

    s_l0, degs = _segsum_deg(x, srcs, dsts, ews)
    d0 = degs[:N].reshape(N, 1)
    d1 = degs[N:].reshape(N, 1)
    h = _layer_call(x, s_l0[0], s_l0[1], d0, d1,
                    wk0, bk0, ws00, ws01, wn00, wn01, b00, b01)
    s_l1 = _segsum_nodeg(h, srcs, dsts, ews)
    return _layer_head_call(h, s_l1[0], s_l1[1], d0, d1,
                            wk1, bk1, ws10, ws11, wn10, wn11, b10, b11,
                            wc1, bc1, wc2, bc2)
